# baseline (speedup 1.0000x reference)
import functools

import jax
import jax.numpy as jnp
from jax import lax
from jax.experimental import pallas as pl
from jax.experimental.pallas import tpu as pltpu
from jax.experimental.pallas import tpu_sc as plsc

D = 64
L = 16
NC = 2
NS = 16
NW = NC * NS
NBUF = 2
SPLIT = 128


def _build(batch, seq):
    assert batch % NW == 0
    rows_w = batch // NW
    assert rows_w % NBUF == 0 and rows_w >= 2 * NBUF
    s2 = seq - SPLIT

    mesh = plsc.VectorSubcoreMesh(core_axis_name="c", subcore_axis_name="s")

    @functools.partial(
        pl.kernel,
        out_type=jax.ShapeDtypeStruct((batch, seq, D), jnp.float32),
        mesh=mesh,
        scratch_types=[
            pltpu.VMEM((rows_w, seq), jnp.int32),
            pltpu.VMEM((rows_w, seq), jnp.int32),
            pltpu.VMEM((NBUF, seq, D), jnp.float32),
            pltpu.VMEM((NBUF, seq, D), jnp.float32),
            pltpu.VMEM((NBUF, seq, D), jnp.float32),
            pltpu.SemaphoreType.DMA((NBUF,)),
            pltpu.SemaphoreType.DMA((NBUF,)),
        ],
        compiler_params=pltpu.CompilerParams(use_tc_tiling_on_sc=False),
    )
    def kern(tok_idx_hbm, diac_idx_hbm, tok_tab_hbm, diac_tab_hbm, out_hbm,
             it_v, id_v, tr_v, dr_v, ob_v, gsems, wsems):
        wid = lax.axis_index("s") * NC + lax.axis_index("c")
        base = wid * rows_w

        pltpu.sync_copy(tok_idx_hbm.at[pl.ds(base, rows_w)], it_v)
        pltpu.sync_copy(diac_idx_hbm.at[pl.ds(base, rows_w)], id_v)

        def gather_descs(g, b):
            return [
                pltpu.make_async_copy(
                    tok_tab_hbm.at[it_v.at[g, pl.ds(0, SPLIT)]],
                    tr_v.at[b, pl.ds(0, SPLIT)], gsems.at[b]),
                pltpu.make_async_copy(
                    tok_tab_hbm.at[it_v.at[g, pl.ds(SPLIT, s2)]],
                    tr_v.at[b, pl.ds(SPLIT, s2)], gsems.at[b]),
                pltpu.make_async_copy(
                    diac_tab_hbm.at[id_v.at[g, pl.ds(0, SPLIT)]],
                    dr_v.at[b, pl.ds(0, SPLIT)], gsems.at[b]),
                pltpu.make_async_copy(
                    diac_tab_hbm.at[id_v.at[g, pl.ds(SPLIT, s2)]],
                    dr_v.at[b, pl.ds(SPLIT, s2)], gsems.at[b]),
            ]

        def issue_gathers(g, b):
            for d in gather_descs(g, b):
                d.start()

        def wait_gathers(g, b):
            for d in gather_descs(g, b):
                d.wait()

        def write_desc(g, b):
            return pltpu.make_async_copy(
                ob_v.at[b], out_hbm.at[base + g], wsems.at[b])

        def add_group(b):
            @pl.loop(0, seq, unroll=4)
            def _(i):
                for j in range(D // L):
                    s = pl.ds(j * L, L)
                    ob_v[b, i, s] = tr_v[b, i, s] + dr_v[b, i, s]

        for b in range(NBUF):
            issue_gathers(b, b)

        @pl.loop(0, rows_w - NBUF, step=NBUF)
        def _(g0):
            for b in range(NBUF):
                g = g0 + b
                wait_gathers(g, b)

                @pl.when(g0 >= NBUF)
                def _():
                    write_desc(g - NBUF, b).wait()

                add_group(b)
                write_desc(g, b).start()
                issue_gathers(g + NBUF, b)

        for b in range(NBUF):
            g = rows_w - NBUF + b
            wait_gathers(g, b)
            write_desc(g - NBUF, b).wait()
            add_group(b)
            write_desc(g, b).start()
        for b in range(NBUF):
            g = rows_w - NBUF + b
            write_desc(g, b).wait()

    return kern


_kern = _build(4096, 200)


def kernel(token_inputs, diac_inputs, token_table, diac_table):
    return _kern(token_inputs, diac_inputs, token_table, diac_table)
